# baseline (speedup 1.0000x reference)
import functools

import jax
import jax.numpy as jnp
from jax import lax
from jax.experimental import pallas as pl
from jax.experimental.pallas import tpu as pltpu
from jax.experimental.pallas import tpu_sc as plsc

B = 16
S = 4096
D = 1024
HALF = S // 2
LANES = 16
STEPS = HALF // LANES


def _sc_body(view_hbm, a_hat_hbm, a_n_hbm, mask_v, row_v):
    c = lax.axis_index("c")
    s = lax.axis_index("s")

    pltpu.sync_copy(view_hbm.at[s, :, pl.ds(D, LANES)], mask_v)

    lane = lax.iota(jnp.int32, LANES)
    zeros = jnp.zeros((LANES,), jnp.int32)
    ones = jnp.ones((LANES,), jnp.int32)

    def step(i, acc):
        vals = plsc.load_gather(mask_v, [i * LANES + lane, zeros])
        return acc + jnp.where(vals != 0.0, ones, zeros)

    acc = lax.fori_loop(0, STEPS, step, jnp.zeros((LANES,), jnp.int32))
    count = jnp.sum(acc)

    r = count - 2 + c
    r = jnp.where(r < 0, r + S, r)

    pltpu.sync_copy(view_hbm.at[s, r // 2, pl.ds((r % 2) * D, D)], row_v)

    @pl.when(c == 0)
    def _():
        pltpu.sync_copy(row_v, a_hat_hbm.at[s])

    @pl.when(c == 1)
    def _():
        pltpu.sync_copy(row_v, a_n_hbm.at[s])


@jax.jit
def kernel(sent_emb, entity_emb):
    del entity_emb
    view = sent_emb.reshape(B, HALF, 2 * D)

    out_row = jax.ShapeDtypeStruct((B, D), jnp.float32)
    sc_call = pl.kernel(
        _sc_body,
        out_type=(out_row, out_row),
        mesh=plsc.VectorSubcoreMesh(core_axis_name="c", subcore_axis_name="s"),
        scratch_types=[
            pltpu.VMEM((HALF, LANES), jnp.float32),
            pltpu.VMEM((D,), jnp.float32),
        ],
        compiler_params=pltpu.CompilerParams(
            use_tc_tiling_on_sc=False, needs_layout_passes=False
        ),
    )
    sent_a_hat_n, sent_a_n = sc_call(view)
    return sent_a_hat_n, sent_a_n, jnp.asarray(0.0, dtype=jnp.float32)


# speedup vs baseline: 5.5716x; 5.5716x over previous
import jax
import jax.numpy as jnp
from jax import lax
from jax.experimental import pallas as pl
from jax.experimental.pallas import tpu as pltpu
from jax.experimental.pallas import tpu_sc as plsc

B = 16
S = 4096
D = 1024
LANES = 16
CH = 512
N_CH = (S // 2) // CH
STEPS = (CH // 2) // LANES


def _sc_body(sent_hbm, a_hat_hbm, a_n_hbm,
             chunk_v, block_v, acc_v, partner_v, cnt_sh, rows_sh):
    c = lax.axis_index("c")
    s = lax.axis_index("s")
    b = c * 8 + s // 2
    h = s % 2

    lane = lax.iota(jnp.int32, LANES)
    zeros = jnp.zeros((LANES,), jnp.int32)
    ones = jnp.ones((LANES,), jnp.int32)

    acc = jnp.zeros((LANES,), jnp.int32)
    for k in range(N_CH):
        row0 = h * (S // 2) + k * CH
        pltpu.sync_copy(sent_hbm.at[b, pl.ds(row0, CH), pl.ds(0, 128)],
                        chunk_v)

        def step(i, a):
            rows = 2 * (i * LANES + lane) + 1
            vals = plsc.load_gather(chunk_v, [rows, zeros])
            return a + jnp.where(vals != 0.0, ones, zeros)

        acc = lax.fori_loop(0, STEPS, step, acc)

    acc_v[...] = acc
    pltpu.sync_copy(acc_v, cnt_sh.at[s])
    plsc.subcore_barrier()
    pltpu.sync_copy(cnt_sh.at[s ^ 1], partner_v)
    count = jnp.sum(acc_v[...] + partner_v[...])

    r = count - 2 + h
    r = jnp.where(r < 0, r + S, r)
    rb = pl.multiple_of((r // 8) * 8, 8)
    pltpu.sync_copy(sent_hbm.at[b, pl.ds(rb, 8), :], block_v)
    pltpu.sync_copy(block_v.at[r - rb], rows_sh.at[h, s // 2])
    plsc.subcore_barrier()

    out0 = pl.multiple_of(c * 8, 8)

    @pl.when(s == 0)
    def _():
        pltpu.sync_copy(rows_sh.at[0], a_hat_hbm.at[pl.ds(out0, 8), :])

    @pl.when(s == 1)
    def _():
        pltpu.sync_copy(rows_sh.at[1], a_n_hbm.at[pl.ds(out0, 8), :])


@jax.jit
def kernel(sent_emb, entity_emb):
    del entity_emb
    out_row = jax.ShapeDtypeStruct((B, D), jnp.float32)
    sc_call = pl.kernel(
        _sc_body,
        out_type=(out_row, out_row),
        mesh=plsc.VectorSubcoreMesh(core_axis_name="c", subcore_axis_name="s"),
        scratch_types=[
            pltpu.VMEM((CH, 128), jnp.float32),
            pltpu.VMEM((8, D), jnp.float32),
            pltpu.VMEM((LANES,), jnp.int32),
            pltpu.VMEM((LANES,), jnp.int32),
            pltpu.VMEM_SHARED((B, LANES), jnp.int32),
            pltpu.VMEM_SHARED((2, 8, D), jnp.float32),
        ],
        compiler_params=pltpu.CompilerParams(needs_layout_passes=False),
    )
    sent_a_hat_n, sent_a_n = sc_call(sent_emb)
    return sent_a_hat_n, sent_a_n, jnp.asarray(0.0, dtype=jnp.float32)
